# baseline (speedup 1.0000x reference)
import jax
import jax.numpy as jnp
from jax.experimental import pallas as pl
from jax.experimental.pallas import tpu as pltpu
from jax.experimental.pallas import tpu_sc as plsc

_B, _C, _H, _W = 8, 96, 32, 32
_N = _H * _W
_K = 9
_HID = 96
_EPS = 1e-5
_ROWS = 128
_GW = 128
_PAD = 128
_SPLIT = 4


def _mm0(a, b):
    return jax.lax.dot_general(
        a, b, (((0,), (0,)), ((), ())), preferred_element_type=jnp.float32)


def _stage1(x_ref, w1_ref, b1_ref, s1_ref, t1_ref, w2t_ref, w12t_ref,
            gcb_ref, idx_ref, y2_ref, yb_ref, inner_ref, sqi_ref):
    b = pl.program_id(0)
    x = x_ref[0]
    h = jnp.dot(w1_ref[...], x, preferred_element_type=jnp.float32)
    h = h + b1_ref[...]
    h = h * s1_ref[...] + t1_ref[...]
    ht = h.T
    sq_i = jnp.sum(ht * ht, axis=1, keepdims=True)
    sq_j = sq_i.T
    sqi_ref[...] = sq_i
    inner_ref[...] = _mm0(h, h)
    y2 = _mm0(h, w2t_ref[...])
    y2_ref[0] = jnp.concatenate(
        [y2, jnp.zeros((_N, _PAD - _HID), jnp.float32)], axis=1)
    yb_ref[0] = _mm0(h, w12t_ref[...]) + gcb_ref[...]

    def tile_body(t, carry):
        r0 = t * _ROWS
        sqi_t = sqi_ref[pl.ds(r0, _ROWS), :]
        d = (sqi_t - 2.0 * inner_ref[pl.ds(r0, _ROWS), :]) + sq_j
        cols = jax.lax.broadcasted_iota(jnp.int32, (_ROWS, _N), 1)
        ams = []
        for _ in range(_K):
            m = jnp.min(d, axis=1, keepdims=True)
            eq = d == m
            am = jnp.min(jnp.where(eq, cols, _N), axis=1, keepdims=True)
            d = jnp.where(eq, jnp.float32(jnp.inf), d)
            ams.append(am)
        idx_tile = jnp.concatenate(ams, axis=1) + b * _N
        idx_ref[0, pl.ds(r0, _ROWS), :] = idx_tile
        return carry

    jax.lax.fori_loop(0, _N // _ROWS, tile_body, 0)


def _stage3(g_ref, yb_ref, s2_ref, c2_ref, w2f_ref, c3_ref, o_ref):
    g = g_ref[:, :, :_HID]
    node = jnp.max(g, axis=0) + yb_ref[0]
    gn = node * s2_ref[...] + c2_ref[...]
    gn = 0.5 * gn * (1.0 + jax.lax.erf(gn * 0.7071067811865476))
    o = jnp.dot(gn, w2f_ref[...], preferred_element_type=jnp.float32)
    o_ref[0] = o + c3_ref[...]


def _sc_gather(table, inds):
    n_idx = inds.shape[1]

    @pl.kernel(
        out_type=jax.ShapeDtypeStruct((n_idx, _PAD), jnp.float32),
        mesh=plsc.VectorSubcoreMesh(core_axis_name="core",
                                    subcore_axis_name="subcore"),
    )
    def gather_kernel(t_hbm, i_hbm, o_hbm):
        def body(i_vmem, o_vmem):
            pltpu.sync_copy(t_hbm.at[i_vmem.at[0]], o_vmem)

        pltpu.emit_pipeline(
            body,
            grid=(n_idx // _GW,),
            in_specs=[pl.BlockSpec((1, _GW), index_map=lambda i: (0, i))],
            out_specs=[pl.BlockSpec((_GW, _PAD), index_map=lambda i: (i, 0))],
            core_axis_name=("core", "subcore"),
            dimension_semantics=(pltpu.PARALLEL,),
        )(i_hbm, o_hbm)

    return gather_kernel(table, inds)


def _run_batches(xr, fc1_w, b1, s1, t1, w2t, w12t, gcb, s2, c2, w2f, c3):
    nb = xr.shape[0]
    idx, y2, yb = pl.pallas_call(
        _stage1,
        grid=(nb,),
        in_specs=[
            pl.BlockSpec((1, _C, _N), lambda b: (b, 0, 0)),
            pl.BlockSpec((_C, _C), lambda b: (0, 0)),
            pl.BlockSpec((_C, 1), lambda b: (0, 0)),
            pl.BlockSpec((_C, 1), lambda b: (0, 0)),
            pl.BlockSpec((_C, 1), lambda b: (0, 0)),
            pl.BlockSpec((_C, _HID), lambda b: (0, 0)),
            pl.BlockSpec((_C, _HID), lambda b: (0, 0)),
            pl.BlockSpec((1, _HID), lambda b: (0, 0)),
        ],
        out_specs=[
            pl.BlockSpec((1, _N, _K), lambda b: (b, 0, 0)),
            pl.BlockSpec((1, _N, _PAD), lambda b: (b, 0, 0)),
            pl.BlockSpec((1, _N, _HID), lambda b: (b, 0, 0)),
        ],
        out_shape=[
            jax.ShapeDtypeStruct((nb, _N, _K), jnp.int32),
            jax.ShapeDtypeStruct((nb, _N, _PAD), jnp.float32),
            jax.ShapeDtypeStruct((nb, _N, _HID), jnp.float32),
        ],
        scratch_shapes=[pltpu.VMEM((_N, _N), jnp.float32),
                        pltpu.VMEM((_N, 1), jnp.float32)],
    )(xr, fc1_w, b1, s1, t1, w2t, w12t, gcb)

    table = y2.reshape(nb * _N, _PAD)
    inds = jnp.transpose(idx, (2, 0, 1)).reshape(1, _K * nb * _N)
    gathered = _sc_gather(table, inds).reshape(_K, nb * _N, _PAD)

    return pl.pallas_call(
        _stage3,
        grid=(nb,),
        in_specs=[
            pl.BlockSpec((_K, _N, _PAD), lambda b: (0, b, 0)),
            pl.BlockSpec((1, _N, _HID), lambda b: (b, 0, 0)),
            pl.BlockSpec((1, _HID), lambda b: (0, 0)),
            pl.BlockSpec((1, _HID), lambda b: (0, 0)),
            pl.BlockSpec((_HID, _C), lambda b: (0, 0)),
            pl.BlockSpec((1, _C), lambda b: (0, 0)),
        ],
        out_specs=pl.BlockSpec((1, _N, _C), lambda b: (b, 0, 0)),
        out_shape=jax.ShapeDtypeStruct((nb, _N, _C), jnp.float32),
    )(gathered, yb, s2, c2, w2f, c3)


def kernel(x, fc1_w, fc1_b, bn1_g, bn1_b, gc_w, gc_b, bn2_g, bn2_b,
           fc2_w, fc2_b, bn3_g, bn3_b):
    xr = x.reshape(_B, _C, _N)

    s1 = (bn1_g / jnp.sqrt(1.0 + _EPS)).reshape(_C, 1)
    t1 = bn1_b.reshape(_C, 1)
    b1 = fc1_b.reshape(_C, 1)
    bs = 1.0 / jnp.sqrt(1.0 + _EPS)
    w1h = gc_w[:, :_C]
    w2h = gc_w[:, _C:]
    w2t = w2h.T
    w12t = (w1h - w2h).T
    gcb = gc_b.reshape(1, _HID)
    s2 = (bn2_g * bs).reshape(1, _HID)
    c2 = bn2_b.reshape(1, _HID)
    s3 = bn3_g * bs
    w2f = (fc2_w * s3[:, None]).T
    c3 = (fc2_b * s3 + bn3_b).reshape(1, _C)

    consts = (fc1_w, b1, s1, t1, w2t, w12t, gcb, s2, c2, w2f, c3)
    nb = _B // _SPLIT
    outs = [_run_batches(xr[i * nb:(i + 1) * nb], *consts)
            for i in range(_SPLIT)]
    out = jnp.concatenate(outs, axis=0)

    return (jnp.transpose(out, (0, 2, 1)) + xr).reshape(_B, _C, _H, _W)


# speedup vs baseline: 1.0489x; 1.0489x over previous
import jax
import jax.numpy as jnp
from jax.experimental import pallas as pl
from jax.experimental.pallas import tpu as pltpu
from jax.experimental.pallas import tpu_sc as plsc

_B, _C, _H, _W = 8, 96, 32, 32
_N = _H * _W
_K = 9
_HID = 96
_EPS = 1e-5
_ROWS = 128
_GW = 128
_PAD = 128
_SPLIT = 4


def _mm0(a, b):
    return jax.lax.dot_general(
        a, b, (((0,), (0,)), ((), ())), preferred_element_type=jnp.float32)


def _stage1(x_ref, w1_ref, b1_ref, s1_ref, t1_ref, w2t_ref, w12t_ref,
            gcb_ref, idx_ref, y2_ref, yb_ref, inner_ref, sqi_ref):
    b = pl.program_id(0)
    x = x_ref[0]
    h = jnp.dot(w1_ref[...], x, preferred_element_type=jnp.float32)
    h = h + b1_ref[...]
    h = h * s1_ref[...] + t1_ref[...]
    ht = h.T
    sq_i = jnp.sum(ht * ht, axis=1, keepdims=True)
    sq_j = sq_i.T
    sqi_ref[...] = sq_i
    inner_ref[...] = _mm0(h, h)
    y2 = _mm0(h, w2t_ref[...])
    y2_ref[0] = jnp.concatenate(
        [y2, jnp.zeros((_N, _PAD - _HID), jnp.float32)], axis=1)
    yb_ref[0] = _mm0(h, w12t_ref[...]) + gcb_ref[...]

    nb = pl.num_programs(0)

    def tile_body(t, carry):
        r0 = t * _ROWS
        sqi_t = sqi_ref[pl.ds(r0, _ROWS), :]
        d = (sqi_t - 2.0 * inner_ref[pl.ds(r0, _ROWS), :]) + sq_j
        colsf = jax.lax.broadcasted_iota(
            jnp.int32, (_ROWS, _N), 1).astype(jnp.float32)
        ones = jnp.ones((_N, 1), jnp.float32)
        ams = []
        for _ in range(_K):
            m = jnp.min(d, axis=1, keepdims=True)
            eq = d == m
            eqf = jnp.where(eq, colsf, 0.0)
            am = jax.lax.dot_general(
                eqf, ones, (((1,), (0,)), ((), ())),
                preferred_element_type=jnp.float32)
            d = jnp.where(eq, jnp.float32(jnp.inf), d)
            ams.append(am)
        idx_tile = jnp.concatenate(ams, axis=1).astype(jnp.int32) + b * _N
        idx_tile = jnp.minimum(idx_tile, nb * _N - 1)
        idx_ref[0, pl.ds(r0, _ROWS), :] = idx_tile
        return carry

    jax.lax.fori_loop(0, _N // _ROWS, tile_body, 0)


def _stage3(g_ref, yb_ref, s2_ref, c2_ref, w2f_ref, c3_ref, o_ref):
    g = g_ref[:, :, :_HID]
    node = jnp.max(g, axis=0) + yb_ref[0]
    gn = node * s2_ref[...] + c2_ref[...]
    gn = 0.5 * gn * (1.0 + jax.lax.erf(gn * 0.7071067811865476))
    o = jnp.dot(gn, w2f_ref[...], preferred_element_type=jnp.float32)
    o_ref[0] = o + c3_ref[...]


def _sc_gather(table, inds):
    n_idx = inds.shape[1]

    @pl.kernel(
        out_type=jax.ShapeDtypeStruct((n_idx, _PAD), jnp.float32),
        mesh=plsc.VectorSubcoreMesh(core_axis_name="core",
                                    subcore_axis_name="subcore"),
    )
    def gather_kernel(t_hbm, i_hbm, o_hbm):
        def body(i_vmem, o_vmem):
            pltpu.sync_copy(t_hbm.at[i_vmem.at[0]], o_vmem)

        pltpu.emit_pipeline(
            body,
            grid=(n_idx // _GW,),
            in_specs=[pl.BlockSpec((1, _GW), index_map=lambda i: (0, i))],
            out_specs=[pl.BlockSpec((_GW, _PAD), index_map=lambda i: (i, 0))],
            core_axis_name=("core", "subcore"),
            dimension_semantics=(pltpu.PARALLEL,),
        )(i_hbm, o_hbm)

    return gather_kernel(table, inds)


def _run_batches(xr, fc1_w, b1, s1, t1, w2t, w12t, gcb, s2, c2, w2f, c3):
    nb = xr.shape[0]
    idx, y2, yb = pl.pallas_call(
        _stage1,
        grid=(nb,),
        in_specs=[
            pl.BlockSpec((1, _C, _N), lambda b: (b, 0, 0)),
            pl.BlockSpec((_C, _C), lambda b: (0, 0)),
            pl.BlockSpec((_C, 1), lambda b: (0, 0)),
            pl.BlockSpec((_C, 1), lambda b: (0, 0)),
            pl.BlockSpec((_C, 1), lambda b: (0, 0)),
            pl.BlockSpec((_C, _HID), lambda b: (0, 0)),
            pl.BlockSpec((_C, _HID), lambda b: (0, 0)),
            pl.BlockSpec((1, _HID), lambda b: (0, 0)),
        ],
        out_specs=[
            pl.BlockSpec((1, _N, _K), lambda b: (b, 0, 0)),
            pl.BlockSpec((1, _N, _PAD), lambda b: (b, 0, 0)),
            pl.BlockSpec((1, _N, _HID), lambda b: (b, 0, 0)),
        ],
        out_shape=[
            jax.ShapeDtypeStruct((nb, _N, _K), jnp.int32),
            jax.ShapeDtypeStruct((nb, _N, _PAD), jnp.float32),
            jax.ShapeDtypeStruct((nb, _N, _HID), jnp.float32),
        ],
        scratch_shapes=[pltpu.VMEM((_N, _N), jnp.float32),
                        pltpu.VMEM((_N, 1), jnp.float32)],
    )(xr, fc1_w, b1, s1, t1, w2t, w12t, gcb)

    table = y2.reshape(nb * _N, _PAD)
    inds = jnp.transpose(idx, (2, 0, 1)).reshape(1, _K * nb * _N)
    gathered = _sc_gather(table, inds).reshape(_K, nb * _N, _PAD)

    return pl.pallas_call(
        _stage3,
        grid=(nb,),
        in_specs=[
            pl.BlockSpec((_K, _N, _PAD), lambda b: (0, b, 0)),
            pl.BlockSpec((1, _N, _HID), lambda b: (b, 0, 0)),
            pl.BlockSpec((1, _HID), lambda b: (0, 0)),
            pl.BlockSpec((1, _HID), lambda b: (0, 0)),
            pl.BlockSpec((_HID, _C), lambda b: (0, 0)),
            pl.BlockSpec((1, _C), lambda b: (0, 0)),
        ],
        out_specs=pl.BlockSpec((1, _N, _C), lambda b: (b, 0, 0)),
        out_shape=jax.ShapeDtypeStruct((nb, _N, _C), jnp.float32),
    )(gathered, yb, s2, c2, w2f, c3)


def kernel(x, fc1_w, fc1_b, bn1_g, bn1_b, gc_w, gc_b, bn2_g, bn2_b,
           fc2_w, fc2_b, bn3_g, bn3_b):
    xr = x.reshape(_B, _C, _N)

    s1 = (bn1_g / jnp.sqrt(1.0 + _EPS)).reshape(_C, 1)
    t1 = bn1_b.reshape(_C, 1)
    b1 = fc1_b.reshape(_C, 1)
    bs = 1.0 / jnp.sqrt(1.0 + _EPS)
    w1h = gc_w[:, :_C]
    w2h = gc_w[:, _C:]
    w2t = w2h.T
    w12t = (w1h - w2h).T
    gcb = gc_b.reshape(1, _HID)
    s2 = (bn2_g * bs).reshape(1, _HID)
    c2 = bn2_b.reshape(1, _HID)
    s3 = bn3_g * bs
    w2f = (fc2_w * s3[:, None]).T
    c3 = (fc2_b * s3 + bn3_b).reshape(1, _C)

    consts = (fc1_w, b1, s1, t1, w2t, w12t, gcb, s2, c2, w2f, c3)
    nb = _B // _SPLIT
    outs = [_run_batches(xr[i * nb:(i + 1) * nb], *consts)
            for i in range(_SPLIT)]
    out = jnp.concatenate(outs, axis=0)

    return (jnp.transpose(out, (0, 2, 1)) + xr).reshape(_B, _C, _H, _W)
